# baseline (speedup 1.0000x reference)
import jax
import jax.numpy as jnp
from jax import lax
from jax.experimental import pallas as pl
from jax.experimental.pallas import tpu as pltpu

OH, OW = 7, 7
E0, E1, E2, S8, LO, HI = 0, OH, 2 * OH, 3 * OH, 3 * OH + OW, 3 * OH + 2 * OW


def _do_roi(scal_ref, ttt, out_ref, slot, n, accs, trs):
    ci = lax.broadcasted_iota(jnp.int32, (24, 1), 0)
    masks = [
        (ci >= scal_ref[1 + LO + j, n]) & (ci < scal_ref[1 + HI + j, n])
        for j in range(OW)
    ]
    ninf = jnp.float32(-jnp.inf)
    for i in range(OH + 2):
        if i < OH:
            a0 = ttt[scal_ref[1 + E0 + i, n]]
            a1 = ttt[scal_ref[1 + E1 + i, n]]
            a2 = ttt[scal_ref[1 + E2 + i, n]]
            accs[i % 2][0:64] = jnp.maximum(jnp.maximum(a0, a1), a2)
        if i >= 2:
            c = i - 2
            tr0, tr1 = trs[c % 2]
            r0 = tr0[0:64:9]
            r1 = tr1[0:64:9]
            for k in range(1, 8):
                r0 = jnp.maximum(r0, tr0[k : k + 64 : 9])
                r1 = jnp.maximum(r1, tr1[k : k + 64 : 9])
            out_ref[slot, c] = jnp.concatenate([r0[0:OW], r1[0:OW]], axis=1)
        if 1 <= i <= OH:
            b = i - 1
            tr0, tr1 = trs[b % 2]
            for j in range(OW):
                s8 = pl.multiple_of(scal_ref[1 + S8 + j, n], 8)
                chunk = accs[b % 2][pl.ds(s8, 24)]
                z = jnp.where(masks[j], chunk, ninf)
                part = jnp.maximum(jnp.maximum(z[0:8], z[8:16]), z[16:24])
                tr0[9 * j : 9 * j + 8] = part[:, 0:128]
                tr1[9 * j : 9 * j + 8] = part[:, 128:256]


NROI = 8


def _roi_kernel(scal_ref, fm_hbm, out_ref, ttt, *rest):
    sem = rest[-1]
    accs = rest[: 2 * NROI]
    trs = rest[2 * NROI : -1]
    n = pl.program_id(0)

    @pl.when(n == 0)
    def _init():
        pad = jnp.full((16, 256), -jnp.inf, jnp.float32)
        for a in accs:
            a[64:80] = pad
        cp = pltpu.make_async_copy(fm_hbm, ttt.at[0:256], sem)
        cp.start()
        cp.wait()

        def build(dst_base, src_base, d, cap):
            def body(e, carry):
                r = e & 63
                p = e + jnp.where(r <= cap, d, 0)
                ttt[dst_base + e] = jnp.maximum(ttt[src_base + e], ttt[src_base + p])
                return carry

            lax.fori_loop(0, 256, body, 0)

        build(256, 0, 1, 62)
        build(512, 256, 2, 61)

    for r in range(NROI):
        _do_roi(
            scal_ref, ttt, out_ref, r, NROI * n + r,
            accs[2 * r : 2 * r + 2],
            (trs[4 * r : 4 * r + 2], trs[4 * r + 2 : 4 * r + 4]),
        )


def kernel(feature_map, rois):
    B, C, H, W = feature_map.shape
    N = rois.shape[0]
    fm = jnp.transpose(feature_map, (0, 2, 3, 1)).reshape(B * H, W, C)

    coords = rois[:, 1:].astype(jnp.int32) // 16
    idx = jnp.clip(rois[:, 0].astype(jnp.int32), 0, B - 1)
    ltx, lty, rbx, rby = coords[:, 0], coords[:, 1], coords[:, 2], coords[:, 3]
    h_roi = rby - lty + 1
    w_roi = rbx - ltx + 1
    oi = jnp.arange(OH, dtype=jnp.int32)
    oj = jnp.arange(OW, dtype=jnp.int32)
    hs = lty[:, None] + (oi[None, :] * h_roi[:, None]) // OH
    he = lty[:, None] + -((-(oi[None, :] + 1) * h_roi[:, None]) // OH)
    ws = ltx[:, None] + (oj[None, :] * w_roi[:, None]) // OW
    we = ltx[:, None] + -((-(oj[None, :] + 1) * w_roi[:, None]) // OW)
    hs = jnp.clip(hs, 0, H - 1)
    he = jnp.clip(he, hs + 1, H)
    ws = jnp.clip(ws, 0, W - 1)
    we = jnp.clip(we, 0, W)

    L = he - hs
    t2 = L >= 4
    t1 = (L >= 2) & ~t2
    r1 = jnp.where(t2, hs + (L - 4) // 2, jnp.where(t1, hs + L - 2, hs))
    r2 = jnp.where(t2, hs + L - 4, jnp.where(t1, hs + L - 2, hs))
    lvl = jnp.where(t2, 2, jnp.where(t1, 1, 0))
    base = lvl * (B * H) + idx[:, None] * H
    e0 = base + hs
    e1 = base + r1
    e2 = base + r2

    s8 = (ws >> 3) << 3
    lo = ws - s8
    hi = we - s8
    scal = jnp.concatenate(
        [jnp.zeros((N, 1), jnp.int32), e0, e1, e2, s8, lo, hi], axis=1
    ).astype(jnp.int32).T

    out = pl.pallas_call(
        _roi_kernel,
        grid_spec=pltpu.PrefetchScalarGridSpec(
            num_scalar_prefetch=1,
            grid=(N // NROI,),
            in_specs=[pl.BlockSpec(memory_space=pl.ANY)],
            out_specs=pl.BlockSpec((NROI, OH, OW, C), lambda n, s: (n, 0, 0, 0)),
            scratch_shapes=(
                [pltpu.VMEM((3 * B * H, W, C), jnp.float32)]
                + [pltpu.VMEM((80, 256), jnp.float32) for _ in range(2 * NROI)]
                + [pltpu.VMEM((72, 128), jnp.float32) for _ in range(4 * NROI)]
                + [pltpu.SemaphoreType.DMA]
            ),
        ),
        out_shape=jax.ShapeDtypeStruct((N, OH, OW, C), jnp.float32),
        compiler_params=pltpu.CompilerParams(
            dimension_semantics=("arbitrary",),
            vmem_limit_bytes=58 * 1024 * 1024,
        ),
        name="roi_maxpool",
    )(scal, fm)
    return jnp.transpose(out, (0, 3, 1, 2))


# speedup vs baseline: 1.0086x; 1.0086x over previous
import jax
import jax.numpy as jnp
from jax import lax
from jax.experimental import pallas as pl
from jax.experimental.pallas import tpu as pltpu

OH, OW = 7, 7
E0, E1, E2, S8, LO, HI = 0, OH, 2 * OH, 3 * OH, 3 * OH + OW, 3 * OH + 2 * OW


def _do_roi(scal_ref, ttt, out_ref, slot, n, accs, trs):
    ci = lax.broadcasted_iota(jnp.int32, (24, 1), 0)
    masks = [
        (ci >= scal_ref[1 + LO + j, n]) & (ci < scal_ref[1 + HI + j, n])
        for j in range(OW)
    ]
    ninf = jnp.float32(-jnp.inf)
    for i in range(OH + 2):
        if i < OH:
            a0 = ttt[scal_ref[1 + E0 + i, n]]
            a1 = ttt[scal_ref[1 + E1 + i, n]]
            a2 = ttt[scal_ref[1 + E2 + i, n]]
            accs[i % 2][0:64] = jnp.maximum(jnp.maximum(a0, a1), a2)
        if i >= 2:
            c = i - 2
            tr0, tr1 = trs[c % 2]
            r0 = tr0[0:64:9]
            r1 = tr1[0:64:9]
            for k in range(1, 8):
                r0 = jnp.maximum(r0, tr0[k : k + 64 : 9])
                r1 = jnp.maximum(r1, tr1[k : k + 64 : 9])
            out_ref[slot, c] = jnp.concatenate([r0[0:OW], r1[0:OW]], axis=1)
        if 1 <= i <= OH:
            b = i - 1
            tr0, tr1 = trs[b % 2]
            for j in range(OW):
                s8 = pl.multiple_of(scal_ref[1 + S8 + j, n], 8)
                chunk = accs[b % 2][pl.ds(s8, 24)]
                z = jnp.where(masks[j], chunk, ninf)
                part = jnp.maximum(jnp.maximum(z[0:8], z[8:16]), z[16:24])
                tr0[9 * j : 9 * j + 8] = part[:, 0:128]
                tr1[9 * j : 9 * j + 8] = part[:, 128:256]


NROI = 4


def _roi_kernel(scal_ref, fm_hbm, out_ref, ttt, *rest):
    sem = rest[-1]
    accs = rest[: 2 * NROI]
    trs = rest[2 * NROI : -1]
    n = pl.program_id(0)

    @pl.when(n == 0)
    def _init():
        pad = jnp.full((16, 256), -jnp.inf, jnp.float32)
        for a in accs:
            a[64:80] = pad
        cp = pltpu.make_async_copy(fm_hbm, ttt.at[0:256], sem)
        cp.start()
        cp.wait()

        def build(dst_base, src_base, d, cap):
            def body(e, carry):
                r = e & 63
                p = e + jnp.where(r <= cap, d, 0)
                ttt[dst_base + e] = jnp.maximum(ttt[src_base + e], ttt[src_base + p])
                return carry

            lax.fori_loop(0, 256, body, 0)

        build(256, 0, 1, 62)
        build(512, 256, 2, 61)

    for r in range(NROI):
        _do_roi(
            scal_ref, ttt, out_ref, r, NROI * n + r,
            accs[2 * r : 2 * r + 2],
            (trs[4 * r : 4 * r + 2], trs[4 * r + 2 : 4 * r + 4]),
        )


def kernel(feature_map, rois):
    B, C, H, W = feature_map.shape
    N = rois.shape[0]
    fm = jnp.transpose(feature_map, (0, 2, 3, 1)).reshape(B * H, W, C)

    coords = rois[:, 1:].astype(jnp.int32) // 16
    idx = jnp.clip(rois[:, 0].astype(jnp.int32), 0, B - 1)
    ltx, lty, rbx, rby = coords[:, 0], coords[:, 1], coords[:, 2], coords[:, 3]
    h_roi = rby - lty + 1
    w_roi = rbx - ltx + 1
    oi = jnp.arange(OH, dtype=jnp.int32)
    oj = jnp.arange(OW, dtype=jnp.int32)
    hs = lty[:, None] + (oi[None, :] * h_roi[:, None]) // OH
    he = lty[:, None] + -((-(oi[None, :] + 1) * h_roi[:, None]) // OH)
    ws = ltx[:, None] + (oj[None, :] * w_roi[:, None]) // OW
    we = ltx[:, None] + -((-(oj[None, :] + 1) * w_roi[:, None]) // OW)
    hs = jnp.clip(hs, 0, H - 1)
    he = jnp.clip(he, hs + 1, H)
    ws = jnp.clip(ws, 0, W - 1)
    we = jnp.clip(we, 0, W)

    L = he - hs
    t2 = L >= 4
    t1 = (L >= 2) & ~t2
    r1 = jnp.where(t2, hs + (L - 4) // 2, jnp.where(t1, hs + L - 2, hs))
    r2 = jnp.where(t2, hs + L - 4, jnp.where(t1, hs + L - 2, hs))
    lvl = jnp.where(t2, 2, jnp.where(t1, 1, 0))
    base = lvl * (B * H) + idx[:, None] * H
    e0 = base + hs
    e1 = base + r1
    e2 = base + r2

    s8 = (ws >> 3) << 3
    lo = ws - s8
    hi = we - s8
    scal = jnp.concatenate(
        [jnp.zeros((N, 1), jnp.int32), e0, e1, e2, s8, lo, hi], axis=1
    ).astype(jnp.int32).T

    out = pl.pallas_call(
        _roi_kernel,
        grid_spec=pltpu.PrefetchScalarGridSpec(
            num_scalar_prefetch=1,
            grid=(N // NROI,),
            in_specs=[pl.BlockSpec(memory_space=pl.ANY)],
            out_specs=pl.BlockSpec((NROI, OH, OW, C), lambda n, s: (n, 0, 0, 0)),
            scratch_shapes=(
                [pltpu.VMEM((3 * B * H, W, C), jnp.float32)]
                + [pltpu.VMEM((80, 256), jnp.float32) for _ in range(2 * NROI)]
                + [pltpu.VMEM((72, 128), jnp.float32) for _ in range(4 * NROI)]
                + [pltpu.SemaphoreType.DMA]
            ),
        ),
        out_shape=jax.ShapeDtypeStruct((N, OH, OW, C), jnp.float32),
        compiler_params=pltpu.CompilerParams(
            dimension_semantics=("arbitrary",),
            vmem_limit_bytes=56 * 1024 * 1024,
        ),
        name="roi_maxpool",
    )(scal, fm)
    return jnp.transpose(out, (0, 3, 1, 2))
